# baseline (speedup 1.0000x reference)
import jax
import jax.numpy as jnp
from jax import lax
from jax.experimental import pallas as pl
from jax.experimental.pallas import tpu as pltpu
from jax.experimental.pallas import tpu_sc as plsc

NC = 2
NS = 16
L = 16
NW = NC * NS
B = 16384
D = 32
BPW = B // NW
CH = 8
NIT = BPW // (2 * CH)
NR = BPW // 128


def _body(uidx_hbm, pidx_hbm, nidx_hbm, ut_hbm, dt_hbm,
          pos_out_hbm, neg_out_hbm,
          uidx, pidx, nidx, cols, urows, prows, nrows, pos_v, neg_v,
          semA, semB):
    wid = lax.axis_index("s") * NC + lax.axis_index("c")

    pltpu.sync_copy(uidx_hbm.at[pl.ds(wid * NR, NR)], uidx)
    pltpu.sync_copy(pidx_hbm.at[pl.ds(wid * NR, NR)], pidx)
    pltpu.sync_copy(nidx_hbm.at[pl.ds(wid * NR, NR)], nidx)

    lane = lax.iota(jnp.int32, L)

    def chunk_vec(idx2d, c2):
        row = c2 * L // 128
        col = pl.multiple_of((c2 * L) % 128, L)
        return idx2d[row, pl.ds(col, L)]

    def fire(table, buf, sem, iv, lo):
        for k in range(CH):
            i = iv[lo + k]
            c0 = pl.multiple_of((i >> 7) << 7, 128)
            for jb in range(4):
                pltpu.async_copy(
                    table.at[pl.ds(jb * 8, 8), pl.ds(c0, 128)],
                    buf.at[k, pl.ds(jb * 8, 8)], sem)

    def drain(buf, sem):
        for k in range(CH):
            pltpu.make_async_copy(
                ut_hbm.at[:, pl.ds(0, 128)], buf.at[k], sem).wait()


    def extract(buf, rows, iv, lo, gbase):
        for k in range(CH):
            im = iv[lo + k] & 127
            cidx = jnp.zeros((L,), jnp.int32) + im
            r0 = plsc.load_gather(buf.at[k], [lane, cidx])
            r1 = plsc.load_gather(buf.at[k], [lane + L, cidx])
            off = pl.multiple_of((gbase + k) * D, D)
            rows[pl.ds(off, L)] = r0
            rows[pl.ds(off + L, L)] = r1

    fire(ut_hbm, cols.at[0], semA, chunk_vec(uidx, 0), 0)

    @pl.loop(0, NIT)
    def _(c2):
        gb = c2 * L
        ivu = chunk_vec(uidx, c2)
        ivp = chunk_vec(pidx, c2)
        ivn = chunk_vec(nidx, c2)
        fire(ut_hbm, cols.at[1], semB, ivu, CH)
        drain(cols.at[0], semA)
        extract(cols.at[0], urows, ivu, 0, gb)
        fire(dt_hbm, cols.at[0], semA, ivp, 0)
        drain(cols.at[1], semB)
        extract(cols.at[1], urows, ivu, CH, gb + CH)
        fire(dt_hbm, cols.at[1], semB, ivp, CH)
        drain(cols.at[0], semA)
        extract(cols.at[0], prows, ivp, 0, gb)
        fire(dt_hbm, cols.at[0], semA, ivn, 0)
        drain(cols.at[1], semB)
        extract(cols.at[1], prows, ivp, CH, gb + CH)
        fire(dt_hbm, cols.at[1], semB, ivn, CH)
        drain(cols.at[0], semA)
        extract(cols.at[0], nrows, ivn, 0, gb)
        @pl.when(c2 < NIT - 1)
        def _():
            fire(ut_hbm, cols.at[0], semA, chunk_vec(uidx, c2 + 1), 0)
        drain(cols.at[1], semB)
        extract(cols.at[1], nrows, ivn, CH, gb + CH)

    @pl.loop(0, BPW // L)
    def _(g):
        rowsD = (g * L + lane) * D
        pacc = jnp.zeros((L,), jnp.float32)
        nacc = jnp.zeros((L,), jnp.float32)
        for j in range(D):
            uv = plsc.load_gather(urows, [rowsD + j])
            pv = plsc.load_gather(prows, [rowsD + j])
            nv = plsc.load_gather(nrows, [rowsD + j])
            pacc = pacc + uv * pv
            nacc = nacc + uv * nv
        off = pl.multiple_of(g * L, L)
        pos_v[pl.ds(off, L)] = pacc
        neg_v[pl.ds(off, L)] = nacc

    base = pl.multiple_of(wid * BPW, BPW)
    pltpu.sync_copy(pos_v, pos_out_hbm.at[pl.ds(base, BPW)])
    pltpu.sync_copy(neg_v, neg_out_hbm.at[pl.ds(base, BPW)])


def kernel(input_users, input_pos_items, input_neg_items, user_vec, doc_vec):
    u2 = input_users.astype(jnp.int32).reshape(B // 128, 128)
    p2 = input_pos_items.astype(jnp.int32).reshape(B // 128, 128)
    n2 = input_neg_items.astype(jnp.int32).reshape(B // 128, 128)

    mesh = plsc.VectorSubcoreMesh(
        core_axis_name="c", subcore_axis_name="s",
        num_cores=NC, num_subcores=NS)

    f = pl.kernel(
        _body,
        out_type=(jax.ShapeDtypeStruct((B,), jnp.float32),
                  jax.ShapeDtypeStruct((B,), jnp.float32)),
        mesh=mesh,
        compiler_params=pltpu.CompilerParams(
            needs_layout_passes=False, use_tc_tiling_on_sc=True),
        scratch_types=[
            pltpu.VMEM((NR, 128), jnp.int32),
            pltpu.VMEM((NR, 128), jnp.int32),
            pltpu.VMEM((NR, 128), jnp.int32),
            pltpu.VMEM((2, CH, D, 128), jnp.float32),
            pltpu.VMEM((BPW * D,), jnp.float32),
            pltpu.VMEM((BPW * D,), jnp.float32),
            pltpu.VMEM((BPW * D,), jnp.float32),
            pltpu.VMEM((BPW,), jnp.float32),
            pltpu.VMEM((BPW,), jnp.float32),
            pltpu.SemaphoreType.DMA,
            pltpu.SemaphoreType.DMA,
        ],
    )
    pos_scores, neg_scores = f(u2, p2, n2, user_vec.T, doc_vec.T)
    return (pos_scores, neg_scores)


# speedup vs baseline: 1.0109x; 1.0109x over previous
import jax
import jax.numpy as jnp
from jax import lax
from jax.experimental import pallas as pl
from jax.experimental.pallas import tpu as pltpu
from jax.experimental.pallas import tpu_sc as plsc

NC = 2
NS = 16
L = 16
NW = NC * NS
B = 16384
D = 32
BPW = B // NW
CH = 8
NIT = BPW // (2 * CH)
NR = BPW // 128


def _body(uidx_hbm, pidx_hbm, nidx_hbm, ut_hbm, dt_hbm,
          pos_out_hbm, neg_out_hbm,
          uidx, pidx, nidx, cols, urows, prows, nrows, pos_v, neg_v,
          semA, semB):
    wid = lax.axis_index("s") * NC + lax.axis_index("c")

    pltpu.sync_copy(uidx_hbm.at[pl.ds(wid * NR, NR)], uidx)
    pltpu.sync_copy(pidx_hbm.at[pl.ds(wid * NR, NR)], pidx)
    pltpu.sync_copy(nidx_hbm.at[pl.ds(wid * NR, NR)], nidx)

    lane = lax.iota(jnp.int32, L)

    def chunk_vec(idx2d, c2):
        row = c2 * L // 128
        col = pl.multiple_of((c2 * L) % 128, L)
        return idx2d[row, pl.ds(col, L)]

    def fire(table, buf, sem, iv, lo):
        for k in range(CH):
            i = iv[lo + k]
            c0 = pl.multiple_of((i >> 7) << 7, 128)
            pltpu.async_copy(table.at[:, pl.ds(c0, 128)], buf.at[k], sem)

    def drain(buf, sem):
        for k in range(CH):
            pltpu.make_async_copy(
                ut_hbm.at[:, pl.ds(0, 128)], buf.at[k], sem).wait()

    def extract(buf, rows, iv, lo, gbase):
        for k in range(CH):
            im = iv[lo + k] & 127
            cidx = jnp.zeros((L,), jnp.int32) + im
            r0 = plsc.load_gather(buf.at[k], [lane, cidx])
            r1 = plsc.load_gather(buf.at[k], [lane + L, cidx])
            off = pl.multiple_of((gbase + k) * D, D)
            rows[pl.ds(off, L)] = r0
            rows[pl.ds(off + L, L)] = r1

    fire(ut_hbm, cols.at[0], semA, chunk_vec(uidx, 0), 0)

    @pl.loop(0, NIT)
    def _(c2):
        gb = c2 * L
        ivu = chunk_vec(uidx, c2)
        ivp = chunk_vec(pidx, c2)
        ivn = chunk_vec(nidx, c2)
        fire(ut_hbm, cols.at[1], semB, ivu, CH)
        drain(cols.at[0], semA)
        extract(cols.at[0], urows, ivu, 0, gb)
        fire(dt_hbm, cols.at[0], semA, ivp, 0)
        drain(cols.at[1], semB)
        extract(cols.at[1], urows, ivu, CH, gb + CH)
        fire(dt_hbm, cols.at[1], semB, ivp, CH)
        drain(cols.at[0], semA)
        extract(cols.at[0], prows, ivp, 0, gb)
        fire(dt_hbm, cols.at[0], semA, ivn, 0)
        drain(cols.at[1], semB)
        extract(cols.at[1], prows, ivp, CH, gb + CH)
        fire(dt_hbm, cols.at[1], semB, ivn, CH)
        drain(cols.at[0], semA)
        extract(cols.at[0], nrows, ivn, 0, gb)
        @pl.when(c2 < NIT - 1)
        def _():
            fire(ut_hbm, cols.at[0], semA, chunk_vec(uidx, c2 + 1), 0)
        drain(cols.at[1], semB)
        extract(cols.at[1], nrows, ivn, CH, gb + CH)

    @pl.loop(0, BPW // L)
    def _(g):
        rowsD = (g * L + lane) * D
        pacc = jnp.zeros((L,), jnp.float32)
        nacc = jnp.zeros((L,), jnp.float32)
        for j in range(D):
            uv = plsc.load_gather(urows, [rowsD + j])
            pv = plsc.load_gather(prows, [rowsD + j])
            nv = plsc.load_gather(nrows, [rowsD + j])
            pacc = pacc + uv * pv
            nacc = nacc + uv * nv
        off = pl.multiple_of(g * L, L)
        pos_v[pl.ds(off, L)] = pacc
        neg_v[pl.ds(off, L)] = nacc

    base = pl.multiple_of(wid * BPW, BPW)
    pltpu.sync_copy(pos_v, pos_out_hbm.at[pl.ds(base, BPW)])
    pltpu.sync_copy(neg_v, neg_out_hbm.at[pl.ds(base, BPW)])


def kernel(input_users, input_pos_items, input_neg_items, user_vec, doc_vec):
    u2 = input_users.astype(jnp.int32).reshape(B // 128, 128)
    p2 = input_pos_items.astype(jnp.int32).reshape(B // 128, 128)
    n2 = input_neg_items.astype(jnp.int32).reshape(B // 128, 128)

    mesh = plsc.VectorSubcoreMesh(
        core_axis_name="c", subcore_axis_name="s",
        num_cores=NC, num_subcores=NS)

    f = pl.kernel(
        _body,
        out_type=(jax.ShapeDtypeStruct((B,), jnp.float32),
                  jax.ShapeDtypeStruct((B,), jnp.float32)),
        mesh=mesh,
        compiler_params=pltpu.CompilerParams(
            needs_layout_passes=False, use_tc_tiling_on_sc=True),
        scratch_types=[
            pltpu.VMEM((NR, 128), jnp.int32),
            pltpu.VMEM((NR, 128), jnp.int32),
            pltpu.VMEM((NR, 128), jnp.int32),
            pltpu.VMEM((2, CH, D, 128), jnp.float32),
            pltpu.VMEM((BPW * D,), jnp.float32),
            pltpu.VMEM((BPW * D,), jnp.float32),
            pltpu.VMEM((BPW * D,), jnp.float32),
            pltpu.VMEM((BPW,), jnp.float32),
            pltpu.VMEM((BPW,), jnp.float32),
            pltpu.SemaphoreType.DMA,
            pltpu.SemaphoreType.DMA,
        ],
    )
    pos_scores, neg_scores = f(u2, p2, n2, user_vec.T, doc_vec.T)
    return (pos_scores, neg_scores)
